# baseline (speedup 1.0000x reference)
import math

import jax
import jax.numpy as jnp
import numpy as np
from jax import lax
from jax.experimental import pallas as pl

N = 1024
D = 256
A = 14
L = 4
H = 8
DH = D // H
KNB = 64
FF = 4 * D
BINS = 64

BR_A = 16
BR_C = 256

_INF = np.float32(np.inf)
_NEG = np.float32(-1e9)
_INF_UKEY = np.uint32(0xFF800000)


def _ln(x, s, b):
    mu = x.mean(-1, keepdims=True)
    var = ((x - mu) ** 2).mean(-1, keepdims=True)
    return s * (x - mu) / jnp.sqrt(var + 1e-5) + b


def _static_dist_body(disto_ref, resi_r, resi_c, chain_r, chain_c,
                      batch_r, batch_c, px_r, px_c, py_r, py_c, pz_r, pz_c,
                      out_ref):
    d = disto_ref[...]
    e = d
    step = np.float32(22.0 / BINS)
    centers = (lax.broadcasted_iota(jnp.int32, (1, 1, BINS), 2).astype(jnp.float32)
               * step + step * 0.5)
    s = jnp.sum(e, axis=-1)
    w = jnp.sum(e * centers, axis=-1)
    mean_d = w / s
    d_disto = jnp.where(mean_d < 8.0, mean_d, _INF)

    same_batch = batch_r[...] == batch_c[...]
    same_chain = jnp.logical_and(chain_r[...] == chain_c[...], same_batch)
    d_chain = jnp.where(same_chain, jnp.abs(resi_r[...] - resi_c[...]) * 3.81, _INF)
    dx = px_r[...] - px_c[...]
    dy = py_r[...] - py_c[...]
    dz = pz_r[...] - pz_c[...]
    d_pca = jnp.sqrt(dx * dx + dy * dy + dz * dz + 1e-12)

    sd = jnp.minimum(jnp.minimum(d_chain, d_disto), d_pca)
    out_ref[...] = jnp.where(same_batch, sd, _INF)


def _qkv_body(local_ref, ln1s, ln1b, wqkv_ref, out_ref):
    x = _ln(local_ref[...], ln1s[...], ln1b[...])
    out_ref[...] = jnp.dot(x, wqkv_ref[...], preferred_element_type=jnp.float32)


def _layer_body(static_ref, gum_ref, batch_r, batch_c, mask_r, mask_c,
                cx_r, cx_c, cy_r, cy_c, cz_r, cz_c,
                q_ref, k_ref, v_ref, local_ref, pos_ref,
                wo_ref, w1_ref, w2_ref, wpos_ref,
                ln2s, ln2b, ln3s, ln3b,
                local_out, pos_out):
    dx = cx_r[...] - cx_c[...]
    dy = cy_r[...] - cy_c[...]
    dz = cz_r[...] - cz_c[...]
    d_ca = jnp.sqrt(dx * dx + dy * dy + dz * dz + 1e-12)
    dist = jnp.minimum(static_ref[...], d_ca)
    u01 = gum_ref[...]
    g = -jnp.log(-jnp.log(u01 + 1e-06) + 1e-06)
    valid = (batch_r[...] == batch_c[...]) & (mask_r[...] > 0) & (mask_c[...] > 0)
    rd = jnp.where(valid & (g == g), 3.0 * dist - g, _INF)

    u = lax.bitcast_convert_type(rd, jnp.uint32)
    flip = jnp.where(u >> 31 != 0, np.uint32(0xFFFFFFFF), np.uint32(0x80000000))
    ukey = u ^ flip
    ans = jnp.zeros((BR_C, 1), jnp.uint32)
    kk = np.float32(KNB)
    for b in range(31, -1, -1):
        cand = ans + np.uint32((1 << b) - 1)
        cnt = jnp.sum(jnp.where(ukey <= cand, 1.0, 0.0), axis=-1, keepdims=True)
        ans = jnp.where(cnt >= kk, ans, ans + np.uint32(1 << b))
    sel = (ukey <= ans) & (ukey < _INF_UKEY)

    q = q_ref[...]
    kf = k_ref[...]
    vf = v_ref[...]
    scale = np.float32(1.0 / math.sqrt(DH))
    outs = []
    for h in range(H):
        qh = q[:, h * DH:(h + 1) * DH]
        kh = kf[:, h * DH:(h + 1) * DH]
        vh = vf[:, h * DH:(h + 1) * DH]
        lg = lax.dot_general(qh, kh, (((1,), (1,)), ((), ())),
                             preferred_element_type=jnp.float32) * scale
        lg = jnp.where(sel, lg, _NEG)
        e = jnp.exp(lg)
        p = e / jnp.sum(e, axis=-1, keepdims=True)
        outs.append(lax.dot_general(p, vh, (((1,), (0,)), ((), ())),
                                    preferred_element_type=jnp.float32))
    o = jnp.concatenate(outs, axis=-1)

    mrow = mask_r[...]
    loc = local_ref[...] + jnp.dot(o, wo_ref[...],
                                   preferred_element_type=jnp.float32) * mrow
    y = _ln(loc, ln2s[...], ln2b[...])
    ffh = jax.nn.gelu(jnp.dot(y, w1_ref[...], preferred_element_type=jnp.float32))
    loc = loc + jnp.dot(ffh, w2_ref[...], preferred_element_type=jnp.float32) * mrow
    z = _ln(loc, ln3s[...], ln3b[...])
    dpos = jnp.dot(z, wpos_ref[...], preferred_element_type=jnp.float32)
    local_out[...] = loc
    pos_out[...] = pos_ref[...] + 0.1 * dpos * mrow


def _row_spec(w):
    return pl.BlockSpec((BR_C, w), lambda r: (r, 0))


def _full_spec(shape):
    nd = len(shape)
    return pl.BlockSpec(shape, lambda r: (0,) * nd)


def kernel(local, pos, prev_distogram, prev_pos, resi, chain, batch, mask, params):
    f32 = jnp.float32
    resi_r = resi.astype(f32).reshape(N, 1)
    resi_c = resi.astype(f32).reshape(1, N)
    chain_r = chain.astype(f32).reshape(N, 1)
    chain_c = chain.astype(f32).reshape(1, N)
    batch_r = batch.astype(f32).reshape(N, 1)
    batch_c = batch.astype(f32).reshape(1, N)
    mask_r = mask.astype(f32).reshape(N, 1)
    mask_c = mask.astype(f32).reshape(1, N)
    pca = prev_pos[:, 1, :]
    ppx_r, ppy_r, ppz_r = (pca[:, i].reshape(N, 1) for i in range(3))
    ppx_c, ppy_c, ppz_c = (pca[:, i].reshape(1, N) for i in range(3))

    static = pl.pallas_call(
        _static_dist_body,
        grid=(N // BR_A,),
        in_specs=[
            pl.BlockSpec((BR_A, N, BINS), lambda r: (r, 0, 0)),
            pl.BlockSpec((BR_A, 1), lambda r: (r, 0)),
            _full_spec((1, N)),
            pl.BlockSpec((BR_A, 1), lambda r: (r, 0)),
            _full_spec((1, N)),
            pl.BlockSpec((BR_A, 1), lambda r: (r, 0)),
            _full_spec((1, N)),
            pl.BlockSpec((BR_A, 1), lambda r: (r, 0)),
            _full_spec((1, N)),
            pl.BlockSpec((BR_A, 1), lambda r: (r, 0)),
            _full_spec((1, N)),
            pl.BlockSpec((BR_A, 1), lambda r: (r, 0)),
            _full_spec((1, N)),
        ],
        out_specs=pl.BlockSpec((BR_A, N), lambda r: (r, 0)),
        out_shape=jax.ShapeDtypeStruct((N, N), f32),
    )(prev_distogram, resi_r, resi_c, chain_r, chain_c, batch_r, batch_c,
      ppx_r, ppx_c, ppy_r, ppy_c, ppz_r, ppz_c)

    base_rng = jax.random.key(42)
    u01s = jax.vmap(
        lambda i: jax.random.uniform(jax.random.fold_in(base_rng, i), (N, N))
    )(jnp.arange(L))

    p = params
    wqkv_all = jnp.concatenate([p['Wq'], p['Wk'], p['Wv']], axis=-1)
    loc = local
    pos_flat = pos.reshape(N, A * 3)
    traj = []
    for l in range(L):
        wqkv = wqkv_all[l]
        qkv = pl.pallas_call(
            _qkv_body,
            grid=(1,),
            in_specs=[_full_spec((N, D)), _full_spec((1, D)), _full_spec((1, D)),
                      _full_spec((D, 3 * D))],
            out_specs=_full_spec((N, 3 * D)),
            out_shape=jax.ShapeDtypeStruct((N, 3 * D), f32),
        )(loc, p['ln1_s'][l].reshape(1, D), p['ln1_b'][l].reshape(1, D), wqkv)

        cx_r = pos_flat[:, 3:4]
        cy_r = pos_flat[:, 4:5]
        cz_r = pos_flat[:, 5:6]
        cx_c, cy_c, cz_c = cx_r.reshape(1, N), cy_r.reshape(1, N), cz_r.reshape(1, N)

        loc, pos_flat = pl.pallas_call(
            _layer_body,
            grid=(N // BR_C,),
            in_specs=[
                _row_spec(N),
                _row_spec(N),
                _row_spec(1), _full_spec((1, N)),
                _row_spec(1), _full_spec((1, N)),
                _row_spec(1), _full_spec((1, N)),
                _row_spec(1), _full_spec((1, N)),
                _row_spec(1), _full_spec((1, N)),
                pl.BlockSpec((BR_C, D), lambda r: (r, 0)),
                pl.BlockSpec((N, D), lambda r: (0, 1)),
                pl.BlockSpec((N, D), lambda r: (0, 2)),
                _row_spec(D),
                _row_spec(A * 3),
                _full_spec((D, D)),
                _full_spec((D, FF)),
                _full_spec((FF, D)),
                _full_spec((D, A * 3)),
                _full_spec((1, D)), _full_spec((1, D)),
                _full_spec((1, D)), _full_spec((1, D)),
            ],
            out_specs=[_row_spec(D), _row_spec(A * 3)],
            out_shape=[jax.ShapeDtypeStruct((N, D), f32),
                       jax.ShapeDtypeStruct((N, A * 3), f32)],
        )(static, u01s[l], batch_r, batch_c, mask_r, mask_c,
          cx_r, cx_c, cy_r, cy_c, cz_r, cz_c,
          qkv, qkv, qkv, loc, pos_flat,
          p['Wo'][l], p['W1'][l], p['W2'][l], p['Wpos'][l],
          p['ln2_s'][l].reshape(1, D), p['ln2_b'][l].reshape(1, D),
          p['ln3_s'][l].reshape(1, D), p['ln3_b'][l].reshape(1, D))
        traj.append(pos_flat.reshape(N, A, 3))

    return loc, pos_flat.reshape(N, A, 3), jnp.stack(traj, axis=0)


# speedup vs baseline: 1.0479x; 1.0479x over previous
import math

import jax
import jax.numpy as jnp
import numpy as np
from jax import lax
from jax.experimental import pallas as pl

N = 1024
D = 256
A = 14
L = 4
H = 8
DH = D // H
KNB = 64
FF = 4 * D
BINS = 64

BR_A = 16
BR_C = 256

_INF = np.float32(np.inf)
_NEG = np.float32(-1e9)
_INF_UKEY = np.uint32(0xFF800000)


def _ln(x, s, b):
    mu = x.mean(-1, keepdims=True)
    var = ((x - mu) ** 2).mean(-1, keepdims=True)
    return s * (x - mu) / jnp.sqrt(var + 1e-5) + b


def _static_dist_body(disto_ref, pr_ref, pc_ref, out_ref):
    d = disto_ref[...]
    e = jnp.exp(d)
    step = np.float32(22.0 / BINS)
    centers = (lax.broadcasted_iota(jnp.int32, (1, 1, BINS), 2).astype(jnp.float32)
               * step + step * 0.5)
    s = jnp.sum(e, axis=-1)
    w = jnp.sum(e * centers, axis=-1)
    mean_d = w / s
    d_disto = jnp.where(mean_d < 8.0, mean_d, _INF)

    pr = pr_ref[...]
    pc = pc_ref[...]
    same_batch = pr[:, 2:3] == pc[2:3, :]
    same_chain = jnp.logical_and(pr[:, 1:2] == pc[1:2, :], same_batch)
    d_chain = jnp.where(same_chain, jnp.abs(pr[:, 0:1] - pc[0:1, :]) * 3.81, _INF)
    dx = pr[:, 4:5] - pc[4:5, :]
    dy = pr[:, 5:6] - pc[5:6, :]
    dz = pr[:, 6:7] - pc[6:7, :]
    d_pca = jnp.sqrt(dx * dx + dy * dy + dz * dz + 1e-12)

    sd = jnp.minimum(jnp.minimum(d_chain, d_disto), d_pca)
    out_ref[...] = jnp.where(same_batch, sd, _INF)


def _qkv_body(local_ref, ln1s, ln1b, wqkv_ref, out_ref):
    x = _ln(local_ref[...], ln1s[0], ln1b[0])
    out_ref[...] = jnp.dot(x, wqkv_ref[0], preferred_element_type=jnp.float32)


def _layer_body(static_ref, u_ref, pr_ref, pc_ref, cac_ref,
                qkv_ref, k_ref, v_ref, local_ref, pos_ref,
                wo_ref, w1_ref, w2_ref, wpos_ref,
                ln2s, ln2b, ln3s, ln3b,
                wqkv_n, ln1s_n, ln1b_n,
                local_out, pos_out, cac_out, qkv_out):
    pr = pr_ref[...]
    pc = pc_ref[...]
    cac = cac_ref[...]
    pos = pos_ref[...]

    dx = pos[:, 3:4] - cac[0:1, :]
    dy = pos[:, 4:5] - cac[1:2, :]
    dz = pos[:, 5:6] - cac[2:3, :]
    d_ca = jnp.sqrt(dx * dx + dy * dy + dz * dz + 1e-12)
    dist = jnp.minimum(static_ref[...], d_ca)
    u01 = u_ref[0]
    g = -jnp.log(-jnp.log(u01 + 1e-06) + 1e-06)
    valid = (pr[:, 2:3] == pc[2:3, :]) & (pr[:, 3:4] > 0) & (pc[3:4, :] > 0)
    rd = jnp.where(valid & (g == g), 3.0 * dist - g, _INF)

    u = lax.bitcast_convert_type(rd, jnp.uint32)
    flip = jnp.where(u >> 31 != 0, np.uint32(0xFFFFFFFF), np.uint32(0x80000000))
    ukey = u ^ flip
    ans = jnp.zeros((BR_C, 1), jnp.uint32)
    kk = np.float32(KNB)
    for b in range(31, -1, -1):
        cand = ans + np.uint32((1 << b) - 1)
        cnt = jnp.sum(jnp.where(ukey <= cand, 1.0, 0.0), axis=-1, keepdims=True)
        ans = jnp.where(cnt >= kk, ans, ans + np.uint32(1 << b))
    sel = (ukey <= ans) & (ukey < _INF_UKEY)

    q = qkv_ref[...]
    kf = k_ref[...]
    vf = v_ref[...]
    scale = np.float32(1.0 / math.sqrt(DH))
    outs = []
    for h in range(H):
        qh = q[:, h * DH:(h + 1) * DH]
        kh = kf[:, h * DH:(h + 1) * DH]
        vh = vf[:, h * DH:(h + 1) * DH]
        lg = lax.dot_general(qh, kh, (((1,), (1,)), ((), ())),
                             preferred_element_type=jnp.float32) * scale
        lg = jnp.where(sel, lg, _NEG)
        e = jnp.exp(lg)
        p = e / jnp.sum(e, axis=-1, keepdims=True)
        outs.append(lax.dot_general(p, vh, (((1,), (0,)), ((), ())),
                                    preferred_element_type=jnp.float32))
    o = jnp.concatenate(outs, axis=-1)

    mrow = pr[:, 3:4]
    loc = local_ref[...] + jnp.dot(o, wo_ref[0],
                                   preferred_element_type=jnp.float32) * mrow
    y = _ln(loc, ln2s[0], ln2b[0])
    ffh = jax.nn.gelu(jnp.dot(y, w1_ref[0], preferred_element_type=jnp.float32))
    loc = loc + jnp.dot(ffh, w2_ref[0], preferred_element_type=jnp.float32) * mrow
    z = _ln(loc, ln3s[0], ln3b[0])
    dpos = jnp.dot(z, wpos_ref[0], preferred_element_type=jnp.float32)
    new_pos = pos + 0.1 * dpos * mrow
    local_out[...] = loc
    pos_out[...] = new_pos
    cac_out[...] = jnp.transpose(new_pos[:, 3:6], (1, 0))

    xn = _ln(loc, ln1s_n[0], ln1b_n[0])
    qkv_out[...] = jnp.dot(xn, wqkv_n[0], preferred_element_type=jnp.float32)


def kernel(local, pos, prev_distogram, prev_pos, resi, chain, batch, mask, params):
    f32 = jnp.float32
    pca = prev_pos[:, 1, :]
    packed_c = jnp.stack([resi.astype(f32), chain.astype(f32), batch.astype(f32),
                          mask.astype(f32), pca[:, 0], pca[:, 1], pca[:, 2],
                          jnp.zeros((N,), f32)], axis=0)
    packed_r = jnp.transpose(packed_c, (1, 0))

    static = pl.pallas_call(
        _static_dist_body,
        grid=(N // BR_A,),
        in_specs=[
            pl.BlockSpec((BR_A, N, BINS), lambda r: (r, 0, 0)),
            pl.BlockSpec((BR_A, 8), lambda r: (r, 0)),
            pl.BlockSpec((8, N), lambda r: (0, 0)),
        ],
        out_specs=pl.BlockSpec((BR_A, N), lambda r: (r, 0)),
        out_shape=jax.ShapeDtypeStruct((N, N), f32),
    )(prev_distogram, packed_r, packed_c)

    base_rng = jax.random.key(42)
    u01s = jax.vmap(
        lambda i: jax.random.uniform(jax.random.fold_in(base_rng, i), (N, N))
    )(jnp.arange(L))

    p = params
    wqkv_all = jnp.concatenate([p['Wq'], p['Wk'], p['Wv']], axis=-1)
    ln1s3 = p['ln1_s'].reshape(L, 1, D)
    ln1b3 = p['ln1_b'].reshape(L, 1, D)
    ln2s3 = p['ln2_s'].reshape(L, 1, D)
    ln2b3 = p['ln2_b'].reshape(L, 1, D)
    ln3s3 = p['ln3_s'].reshape(L, 1, D)
    ln3b3 = p['ln3_b'].reshape(L, 1, D)
    pos_flat = pos.reshape(N, A * 3)
    cac = jnp.transpose(pos_flat[:, 3:6], (1, 0))

    qkv = pl.pallas_call(
        _qkv_body,
        grid=(1,),
        in_specs=[
            pl.BlockSpec((N, D), lambda r: (0, 0)),
            pl.BlockSpec((1, 1, D), lambda r: (0, 0, 0)),
            pl.BlockSpec((1, 1, D), lambda r: (0, 0, 0)),
            pl.BlockSpec((1, D, 3 * D), lambda r: (0, 0, 0)),
        ],
        out_specs=pl.BlockSpec((N, 3 * D), lambda r: (0, 0)),
        out_shape=jax.ShapeDtypeStruct((N, 3 * D), f32),
    )(local, ln1s3, ln1b3, wqkv_all)

    loc = local
    traj = []
    for l in range(L):
        ln = min(l + 1, L - 1)
        loc, pos_flat, cac, qkv = pl.pallas_call(
            _layer_body,
            grid=(N // BR_C,),
            in_specs=[
                pl.BlockSpec((BR_C, N), lambda r: (r, 0)),
                pl.BlockSpec((1, BR_C, N), lambda r, l=l: (l, r, 0)),
                pl.BlockSpec((BR_C, 8), lambda r: (r, 0)),
                pl.BlockSpec((8, N), lambda r: (0, 0)),
                pl.BlockSpec((3, N), lambda r: (0, 0)),
                pl.BlockSpec((BR_C, D), lambda r: (r, 0)),
                pl.BlockSpec((N, D), lambda r: (0, 1)),
                pl.BlockSpec((N, D), lambda r: (0, 2)),
                pl.BlockSpec((BR_C, D), lambda r: (r, 0)),
                pl.BlockSpec((BR_C, A * 3), lambda r: (r, 0)),
                pl.BlockSpec((1, D, D), lambda r, l=l: (l, 0, 0)),
                pl.BlockSpec((1, D, FF), lambda r, l=l: (l, 0, 0)),
                pl.BlockSpec((1, FF, D), lambda r, l=l: (l, 0, 0)),
                pl.BlockSpec((1, D, A * 3), lambda r, l=l: (l, 0, 0)),
                pl.BlockSpec((1, 1, D), lambda r, l=l: (l, 0, 0)),
                pl.BlockSpec((1, 1, D), lambda r, l=l: (l, 0, 0)),
                pl.BlockSpec((1, 1, D), lambda r, l=l: (l, 0, 0)),
                pl.BlockSpec((1, 1, D), lambda r, l=l: (l, 0, 0)),
                pl.BlockSpec((1, D, 3 * D), lambda r, ln=ln: (ln, 0, 0)),
                pl.BlockSpec((1, 1, D), lambda r, ln=ln: (ln, 0, 0)),
                pl.BlockSpec((1, 1, D), lambda r, ln=ln: (ln, 0, 0)),
            ],
            out_specs=[
                pl.BlockSpec((BR_C, D), lambda r: (r, 0)),
                pl.BlockSpec((BR_C, A * 3), lambda r: (r, 0)),
                pl.BlockSpec((3, BR_C), lambda r: (0, r)),
                pl.BlockSpec((BR_C, 3 * D), lambda r: (r, 0)),
            ],
            out_shape=[jax.ShapeDtypeStruct((N, D), f32),
                       jax.ShapeDtypeStruct((N, A * 3), f32),
                       jax.ShapeDtypeStruct((3, N), f32),
                       jax.ShapeDtypeStruct((N, 3 * D), f32)],
        )(static, u01s, packed_r, packed_c, cac,
          qkv, qkv, qkv, loc, pos_flat,
          p['Wo'], p['W1'], p['W2'], p['Wpos'],
          ln2s3, ln2b3, ln3s3, ln3b3,
          wqkv_all, ln1s3, ln1b3)
        traj.append(pos_flat.reshape(N, A, 3))

    return loc, pos_flat.reshape(N, A, 3), jnp.stack(traj, axis=0)
